# Initial kernel scaffold; baseline (speedup 1.0000x reference)
import jax
import jax.numpy as jnp
from jax.experimental import pallas as pl


def kernel(x, edge_index, batch, W0, b0, W1, b1, W2, b2, g0, be0, m0, v0, g1, be1, m1, v1):
    raise NotImplementedError("write your pallas kernel here")


# speedup vs baseline: 10.6243x; 10.6243x over previous
import functools

import jax
import jax.numpy as jnp
from jax import lax
from jax.experimental import pallas as pl
from jax.experimental.pallas import tpu as pltpu
from jax.experimental.pallas import tpu_sc as plsc

N = 10000
E = 320000
F = 128
H = 128
C = 6
G = 64

_IT = False
NC, NS = 2, 16
NW = NC * NS
CH = 128
NCH = -(-E // (NW * CH))
EP = NW * NCH * CH
NPAD = 10240
RPT = NPAD // NS
PAD_DST = N

def _mesh():
  return plsc.VectorSubcoreMesh(
      core_axis_name="c", subcore_axis_name="s", num_cores=NC, num_subcores=NS)


@functools.lru_cache(maxsize=None)
def _make_mp(D):
  @functools.partial(
      pl.kernel,
      mesh=_mesh(),
      compiler_params=pltpu.CompilerParams(use_tc_tiling_on_sc=False),
      out_type=jax.ShapeDtypeStruct((NC, NPAD, D), jnp.float32),
      scratch_types=[
          pltpu.VMEM((CH,), jnp.int32),
          pltpu.VMEM((CH,), jnp.int32),
          pltpu.VMEM((CH, D), jnp.float32),
          pltpu.VMEM((16, D), jnp.float32),
          pltpu.VMEM_SHARED((NPAD, D), jnp.float32),
          pltpu.SemaphoreType.DMA,
      ],
  )
  def mp(u_hbm, srcs_hbm, dsts_hbm, out_hbm, sidx, didx, rows, zbuf, acc, sem):
    cid = lax.axis_index("c")
    sid = lax.axis_index("s")
    tid = cid * NS + sid
    zero = jnp.zeros((16,), jnp.float32)
    for r in range(16):
      for k in range(D // 16):
        zbuf[r, pl.ds(k * 16, 16)] = zero

    @pl.loop(0, RPT // 16)
    def _(b):
      pltpu.sync_copy(zbuf, acc.at[pl.ds(sid * RPT + b * 16, 16)])

    plsc.subcore_barrier()

    @pl.loop(0, NCH)
    def _(j):
      pltpu.sync_copy(srcs_hbm.at[tid, j], sidx)
      pltpu.sync_copy(dsts_hbm.at[tid, j], didx)
      pltpu.async_copy(u_hbm.at[sidx], rows, sem).wait()
      pltpu.sync_copy(rows, acc.at[didx], add=True)

    plsc.subcore_barrier()
    pltpu.sync_copy(acc.at[pl.ds(sid * RPT, RPT)],
                    out_hbm.at[cid, pl.ds(sid * RPT, RPT)])

  return mp


@functools.lru_cache(maxsize=None)
def _make_deg():

  @functools.partial(
      pl.kernel,
      mesh=_mesh(),
      compiler_params=pltpu.CompilerParams(use_tc_tiling_on_sc=False),
      out_type=jax.ShapeDtypeStruct((NC, NPAD, 16), jnp.float32),
      scratch_types=[
          pltpu.VMEM((CH,), jnp.int32),
          pltpu.VMEM((CH, 16), jnp.float32),
          pltpu.VMEM((16, 16), jnp.float32),
          pltpu.VMEM_SHARED((NPAD, 16), jnp.float32),
      ],
  )
  def deg_kernel(dsts_hbm, out_hbm, didx, ones_rows, zbuf, acc):
    cid = lax.axis_index("c")
    sid = lax.axis_index("s")
    tid = cid * NS + sid
    zero = jnp.zeros((16,), jnp.float32)
    one = jnp.ones((16,), jnp.float32)
    for r in range(16):
      zbuf[r, :] = zero
    for r in range(CH):
      ones_rows[r, :] = one

    @pl.loop(0, RPT // 16)
    def _(b):
      pltpu.sync_copy(zbuf, acc.at[pl.ds(sid * RPT + b * 16, 16)])

    plsc.subcore_barrier()

    @pl.loop(0, NCH)
    def _(j):
      pltpu.sync_copy(dsts_hbm.at[tid, j], didx)
      pltpu.sync_copy(ones_rows, acc.at[didx], add=True)

    plsc.subcore_barrier()
    pltpu.sync_copy(acc.at[pl.ds(sid * RPT, RPT)],
                    out_hbm.at[cid, pl.ds(sid * RPT, RPT)])

  return deg_kernel


BR = 1024
NB = NPAD // BR
_P_HIGH = lax.Precision.HIGHEST


def _dinv_block(deg_ref, i):
  d = deg_ref[0, pl.ds(i * BR, BR), 0:1] + deg_ref[1, pl.ds(i * BR, BR), 0:1]
  return lax.rsqrt(d + 1.0)


def _tc_pre_body(x_ref, w_ref, deg_ref, o_ref):
  i = pl.program_id(0)
  dinv = _dinv_block(deg_ref, i)
  xw = jnp.dot(x_ref[...], w_ref[...], precision=_P_HIGH,
               preferred_element_type=jnp.float32)
  o_ref[...] = xw * dinv


def _tc_pre(x, w, deg):
  return pl.pallas_call(
      _tc_pre_body,
      grid=(NB,),
      in_specs=[
          pl.BlockSpec((BR, F), lambda i: (i, 0)),
          pl.BlockSpec((F, H), lambda i: (0, 0)),
          pl.BlockSpec((NC, NPAD, 16), lambda i: (0, 0, 0)),
      ],
      out_specs=pl.BlockSpec((BR, H), lambda i: (i, 0)),
      out_shape=jax.ShapeDtypeStruct((NPAD, H), jnp.float32),
      interpret=_IT,
  )(x, w, deg)


def _tc_post_body(s_ref, u_ref, deg_ref, b_ref, g_ref, be_ref, m_ref, v_ref,
                  w_ref, o_ref):
  i = pl.program_id(0)
  dinv = _dinv_block(deg_ref, i)
  t = (s_ref[0] + s_ref[1] + u_ref[...]) * dinv + b_ref[...]
  t = (t - m_ref[...]) * lax.rsqrt(v_ref[...] + 1e-5) * g_ref[...] + be_ref[...]
  t = jnp.maximum(t, 0.0)
  o_ref[...] = jnp.dot(t, w_ref[...], precision=_P_HIGH,
                       preferred_element_type=jnp.float32) * dinv


def _tc_post(s, u, deg, b, g, be, m, v, w, do):
  vec = lambda a: a.reshape(1, H)
  return pl.pallas_call(
      _tc_post_body,
      grid=(NB,),
      in_specs=[
          pl.BlockSpec((NC, BR, H), lambda i: (0, i, 0)),
          pl.BlockSpec((BR, H), lambda i: (i, 0)),
          pl.BlockSpec((NC, NPAD, 16), lambda i: (0, 0, 0)),
          pl.BlockSpec((1, H), lambda i: (0, 0)),
          pl.BlockSpec((1, H), lambda i: (0, 0)),
          pl.BlockSpec((1, H), lambda i: (0, 0)),
          pl.BlockSpec((1, H), lambda i: (0, 0)),
          pl.BlockSpec((1, H), lambda i: (0, 0)),
          pl.BlockSpec((H, do), lambda i: (0, 0)),
      ],
      out_specs=pl.BlockSpec((BR, do), lambda i: (i, 0)),
      out_shape=jax.ShapeDtypeStruct((NPAD, do), jnp.float32),
      interpret=_IT,
  )(s, u, deg, vec(b), vec(g), vec(be), vec(m), vec(v), w)


def _tc_pool_body(s_ref, u_ref, deg_ref, b_ref, p_ref, o_ref, acc_ref):
  i = pl.program_id(0)
  dinv = _dinv_block(deg_ref, i)
  h3 = (s_ref[0] + s_ref[1] + u_ref[...]) * dinv + b_ref[...]
  col = lax.broadcasted_iota(jnp.int32, (BR, 16), 1)
  hext = jnp.where(col == 15, 1.0, h3)
  contrib = jnp.dot(p_ref[...], hext, precision=_P_HIGH,
                    preferred_element_type=jnp.float32)

  @pl.when(i == 0)
  def _():
    acc_ref[...] = contrib

  @pl.when(i > 0)
  def _():
    acc_ref[...] += contrib

  @pl.when(i == NB - 1)
  def _():
    a = acc_ref[...]
    counts = jnp.maximum(a[:, 15:16], 1.0)
    pooled = a / counts
    cc = lax.broadcasted_iota(jnp.int32, (G, 16), 1)
    mask = cc < C
    mx = jnp.max(jnp.where(mask, pooled, -1e30), axis=1, keepdims=True)
    ex = jnp.where(mask, jnp.exp(pooled - mx), 0.0)
    se = jnp.sum(ex, axis=1, keepdims=True)
    o_ref[...] = pooled - mx - jnp.log(se)


def _tc_pool(s, u, deg, b2p, p):
  return pl.pallas_call(
      _tc_pool_body,
      grid=(NB,),
      in_specs=[
          pl.BlockSpec((NC, BR, 16), lambda i: (0, i, 0)),
          pl.BlockSpec((BR, 16), lambda i: (i, 0)),
          pl.BlockSpec((NC, NPAD, 16), lambda i: (0, 0, 0)),
          pl.BlockSpec((1, 16), lambda i: (0, 0)),
          pl.BlockSpec((G, BR), lambda i: (0, i)),
      ],
      out_specs=pl.BlockSpec((G, 16), lambda i: (0, 0)),
      out_shape=jax.ShapeDtypeStruct((G, 16), jnp.float32),
      scratch_shapes=[pltpu.VMEM((G, 16), jnp.float32)],
      interpret=_IT,
  )(s, u, deg, b2p.reshape(1, 16), p)


def kernel(x, edge_index, batch, W0, b0, W1, b1, W2, b2,
           g0, be0, m0, v0, g1, be1, m1, v1):
  src = jnp.concatenate([edge_index[0], jnp.zeros((EP - E,), jnp.int32)])
  dst = jnp.concatenate(
      [edge_index[1], jnp.full((EP - E,), PAD_DST, jnp.int32)])
  srcs = src.reshape(NW, NCH, CH)
  dsts = dst.reshape(NW, NCH, CH)
  xp = jnp.zeros((NPAD, F), jnp.float32).at[:N].set(x)
  w2p = jnp.zeros((H, 16), jnp.float32).at[:, :C].set(W2)
  b2p = jnp.zeros((16,), jnp.float32).at[:C].set(b2)
  pool = (batch[None, :] == jnp.arange(G, dtype=batch.dtype)[:, None])
  poolp = jnp.zeros((G, NPAD), jnp.float32).at[:, :N].set(
      pool.astype(jnp.float32))

  mp128 = _make_mp(H)
  mp16 = _make_mp(16)

  deg = _make_deg()(dsts)

  u0 = _tc_pre(xp, W0, deg)
  s0 = mp128(u0, srcs, dsts)
  u1 = _tc_post(s0, u0, deg, b0, g0, be0, m0, v0, W1, H)
  s1 = mp128(u1, srcs, dsts)
  u2 = _tc_post(s1, u1, deg, b1, g1, be1, m1, v1, w2p, 16)
  s2 = mp16(u2, srcs, dsts)
  out = _tc_pool(s2, u2, deg, b2p, poolp)
  return out[:, :C]
